# baseline (speedup 1.0000x reference)
import functools

import jax
import jax.numpy as jnp
from jax import lax
from jax.experimental import pallas as pl
from jax.experimental.pallas import tpu as pltpu
from jax.experimental.pallas import tpu_sc as plsc

N = 10000
E = 160000
D = 32
NBOND = 16
NATOM = 100
G = 256

NPAD = 10240
NR4 = NPAD // 4
D4 = 4 * D

NC = 2
NS = 16
NW = NC * NS
CH = 128
K0 = 79
K1 = 0
NCHUNK = NS * (K0 + K1)
EPAD = NCHUNK * CH
TRASH = NPAD - 1
ZR = NPAD // NS

BLK4 = 512
NB4 = NR4 // BLK4

_DOT = dict(preferred_element_type=jnp.float32, precision=lax.Precision.HIGHEST)


def _bd4(w):
    z = jnp.zeros((D, D), jnp.float32)
    rows = [jnp.concatenate([w if jj == j else z for jj in range(4)], axis=1)
            for j in range(4)]
    return jnp.concatenate(rows, axis=0)


def _split_mm(v, w):
    return jnp.dot(v, _bd4(w), **_DOT)


def _prep_body(ids_ref, aemb_ref, abias_ref, bemb_ref, w1_ref, b1_ref,
               w2_ref, b2_ref, x_ref, ab_ref, k1_ref, k2_ref):
    ids = ids_ref[...]
    iota = lax.broadcasted_iota(jnp.int32, (1, NATOM), 1)
    xs, abs_ = [], []
    for j in range(4):
        oh = (ids[:, j:j + 1] == iota).astype(jnp.float32)
        xs.append(jnp.dot(oh, aemb_ref[...], **_DOT))
        abs_.append(jnp.dot(oh, abias_ref[...], **_DOT))
    x_ref[...] = jnp.concatenate(xs, axis=1)
    ab_ref[...] = jnp.concatenate(abs_, axis=1)

    @pl.when(pl.program_id(0) == 0)
    def _():
        k1_ref[...] = jnp.dot(bemb_ref[...], w1_ref[...], **_DOT) + b1_ref[...]
        k2_ref[...] = jnp.dot(bemb_ref[...], w2_ref[...], **_DOT) + b2_ref[...]


def _y_body(x_ref, k3_ref, y_ref):
    y_ref[...] = _split_mm(x_ref[...], k3_ref[0])


def _hy_body(p_ref, x_ref, root_ref, bias_ref, k3_ref, h_ref, y_ref, hs_ref):
    @pl.when(pl.program_id(0) == 0)
    def _():
        for j in range(NB4):
            sl = pl.ds(j * BLK4, BLK4)
            agg = p_ref[0, sl]
            h = jax.nn.relu(agg + _split_mm(x_ref[sl], root_ref[...])
                            + bias_ref[...])
            hs_ref[sl] = h
            h_ref[sl] = h

    y_ref[...] = _split_mm(hs_ref[...], k3_ref[0])


def _final_body(p_ref, h_ref, root_ref, bias_ref, dw_ref, db_ref, ab_ref,
                seg_ref, pool_ref):
    h = h_ref[...]
    agg = p_ref[0]
    x2 = h + jax.nn.relu(agg + _split_mm(h, root_ref[...]) + bias_ref[...])
    zc = jnp.zeros((D, 1), jnp.float32)
    dw4 = jnp.concatenate(
        [jnp.concatenate([dw_ref[...] if jj == j else zc for jj in range(4)],
                         axis=0) for j in range(4)], axis=1)
    o4 = ab_ref[...] + jnp.dot(x2, dw4, **_DOT) + db_ref[...]
    part = jnp.zeros((G, 1), jnp.float32)
    for j in range(4):
        mask = (lax.broadcasted_iota(jnp.int32, (G, BLK4), 0)
                == seg_ref[0, j]).astype(jnp.float32)
        part = part + jnp.dot(mask, o4[:, j:j + 1], **_DOT)

    @pl.when(pl.program_id(0) == 0)
    def _():
        pool_ref[...] = jnp.zeros_like(pool_ref)

    pool_ref[...] += part


NBUF = 4


def _sc_conv_body(y_hbm, gi_hbm, di_hbm, out_hbm, accum, gi_v, di_v, rows_v,
                  buf_v, semg, sems):
    c = lax.axis_index("c")
    s = lax.axis_index("s")

    @pl.when(c == 0)
    def _():
        def zrow(r, carry):
            rows_v[0, r, pl.ds(0, 16)] = jnp.zeros((16,), jnp.float32)
            rows_v[0, r, pl.ds(16, 16)] = jnp.zeros((16,), jnp.float32)
            return carry

        lax.fori_loop(0, CH, zrow, 0)
        for q in range(ZR // CH):
            pltpu.sync_copy(rows_v.at[0], accum.at[pl.ds(s * ZR + q * CH, CH)])

    plsc.subcore_barrier()

    def edge_pipeline(start, kch):
        pltpu.sync_copy(gi_hbm.at[pl.ds(start, kch)], gi_v.at[pl.ds(0, kch)])
        pltpu.sync_copy(di_hbm.at[pl.ds(start, kch)], di_v.at[pl.ds(0, kch)])

        for j in range(NBUF):
            pltpu.async_copy(y_hbm.at[gi_v.at[j]], rows_v.at[j], semg)

        def step(k, carry):
            @pl.when(jnp.logical_and(k >= 1, k - 1 + NBUF < kch))
            def _():
                pltpu.make_async_copy(rows_v.at[(k - 1) % NBUF],
                                      accum.at[di_v.at[k - 1]], sems).wait()
                pltpu.async_copy(y_hbm.at[gi_v.at[k - 1 + NBUF]],
                                 rows_v.at[(k - 1) % NBUF], semg)

            pltpu.make_async_copy(y_hbm.at[gi_v.at[k]],
                                  rows_v.at[k % NBUF], semg).wait()
            pltpu.async_copy(rows_v.at[k % NBUF], accum.at[di_v.at[k]], sems,
                             add=True)
            return carry

        lax.fori_loop(0, kch, step, 0)
        for j in range(NBUF):
            pltpu.make_async_copy(rows_v.at[j], accum.at[di_v.at[0]],
                                  sems).wait()

    @pl.when(c == 0)
    def _():
        edge_pipeline(s * K0, K0)

    plsc.subcore_barrier()

    @pl.when(c == 0)
    def _():
        pltpu.sync_copy(accum.at[pl.ds(s * ZR, ZR)], buf_v)
        pltpu.sync_copy(buf_v, out_hbm.at[0, pl.ds(s * ZR, ZR)])


_sc_conv = pl.kernel(
    _sc_conv_body,
    out_type=jax.ShapeDtypeStruct((1, NPAD, D), jnp.float32),
    mesh=plsc.VectorSubcoreMesh(core_axis_name="c", subcore_axis_name="s",
                                num_cores=NC, num_subcores=NS),
    scratch_types=[
        pltpu.VMEM_SHARED((NPAD, D), jnp.float32),
        pltpu.VMEM((K0, CH), jnp.int32),
        pltpu.VMEM((K0, CH), jnp.int32),
        pltpu.VMEM((NBUF, CH, D), jnp.float32),
        pltpu.VMEM((ZR, D), jnp.float32),
        pltpu.SemaphoreType.DMA,
        pltpu.SemaphoreType.DMA,
    ],
    compiler_params=pltpu.CompilerParams(use_tc_tiling_on_sc=False),
)


_FULL = lambda shape: pl.BlockSpec(shape, lambda *_: tuple(0 for _ in shape))


def _prep_call(ids4, atom_emb, atom_bias_emb, bond_emb, w1, b1, w2, b2):
    return pl.pallas_call(
        _prep_body,
        grid=(NB4,),
        in_specs=[
            pl.BlockSpec((BLK4, 4), lambda n: (n, 0)),
            _FULL((NATOM, D)),
            _FULL((NATOM, 1)),
            _FULL((NBOND, D)),
            _FULL((D, D * D)),
            _FULL((1, D * D)),
            _FULL((D, D * D)),
            _FULL((1, D * D)),
        ],
        out_specs=[
            pl.BlockSpec((BLK4, D4), lambda n: (n, 0)),
            pl.BlockSpec((BLK4, 4), lambda n: (n, 0)),
            _FULL((NBOND, D * D)),
            _FULL((NBOND, D * D)),
        ],
        out_shape=[
            jax.ShapeDtypeStruct((NR4, D4), jnp.float32),
            jax.ShapeDtypeStruct((NR4, 4), jnp.float32),
            jax.ShapeDtypeStruct((NBOND, D * D), jnp.float32),
            jax.ShapeDtypeStruct((NBOND, D * D), jnp.float32),
        ],
    )(ids4, atom_emb, atom_bias_emb, bond_emb, w1, b1, w2, b2)


def _y_call(x4, k3):
    return pl.pallas_call(
        _y_body,
        grid=(NBOND,),
        in_specs=[
            _FULL((NR4, D4)),
            pl.BlockSpec((1, D, D), lambda b: (b, 0, 0)),
        ],
        out_specs=pl.BlockSpec((NR4, D4), lambda b: (b, 0)),
        out_shape=jax.ShapeDtypeStruct((NBOND * NR4, D4), jnp.float32),
    )(x4, k3)


def _hy_call(p4, x4, root, bias4, k3):
    return pl.pallas_call(
        _hy_body,
        grid=(NBOND,),
        in_specs=[
            _FULL((1, NR4, D4)),
            _FULL((NR4, D4)),
            _FULL((D, D)),
            _FULL((1, D4)),
            pl.BlockSpec((1, D, D), lambda b: (b, 0, 0)),
        ],
        out_specs=[
            _FULL((NR4, D4)),
            pl.BlockSpec((NR4, D4), lambda b: (b, 0)),
        ],
        out_shape=[
            jax.ShapeDtypeStruct((NR4, D4), jnp.float32),
            jax.ShapeDtypeStruct((NBOND * NR4, D4), jnp.float32),
        ],
        scratch_shapes=[pltpu.VMEM((NR4, D4), jnp.float32)],
    )(p4, x4, root, bias4, k3)


def _final_call(p4, h4, root, bias4, dw, db, ab4, segs):
    return pl.pallas_call(
        _final_body,
        grid=(NB4,),
        in_specs=[
            pl.BlockSpec((1, BLK4, D4), lambda n: (0, n, 0)),
            pl.BlockSpec((BLK4, D4), lambda n: (n, 0)),
            _FULL((D, D)),
            _FULL((1, D4)),
            _FULL((D, 1)),
            _FULL((1, 1)),
            pl.BlockSpec((BLK4, 4), lambda n: (n, 0)),
            pl.BlockSpec((1, 4, BLK4), lambda n: (n, 0, 0)),
        ],
        out_specs=pl.BlockSpec((G, 1), lambda n: (0, 0)),
        out_shape=jax.ShapeDtypeStruct((G, 1), jnp.float32),
    )(p4, h4, root, bias4, dw, db, ab4, segs)


def kernel(x_in, edge_index, e_in, i, atom_emb, atom_bias_emb, bond_emb,
           conv1_kn_W, conv1_kn_b, conv1_root, conv1_bias,
           conv2_kn_W, conv2_kn_b, conv2_root, conv2_bias,
           dense_W, dense_b):
    eif = edge_index.reshape(2 * E).astype(jnp.int32)
    src = lax.slice(eif, (0,), (E,))
    dst = lax.slice(eif, (E,), (2 * E,))
    bond = e_in.reshape(E).astype(jnp.int32)

    gidx = bond * NPAD + src
    gidx = jnp.concatenate(
        [gidx, jnp.zeros((EPAD - E,), jnp.int32)]).reshape(NCHUNK, CH)
    dsti = jnp.concatenate(
        [dst, jnp.full((EPAD - E,), TRASH, jnp.int32)]).reshape(NCHUNK, CH)

    ids4 = jnp.concatenate(
        [x_in[:, 0], jnp.zeros((NPAD - N,), x_in.dtype)]).reshape(NR4, 4)
    segs = jnp.concatenate(
        [i.astype(jnp.int32), jnp.full((NPAD - N,), G, jnp.int32)]
    ).reshape(NB4, BLK4, 4).transpose(0, 2, 1)

    x4, ab4, k1, k2 = _prep_call(ids4, atom_emb, atom_bias_emb, bond_emb,
                                 conv1_kn_W, conv1_kn_b.reshape(1, D * D),
                                 conv2_kn_W, conv2_kn_b.reshape(1, D * D))
    k13 = k1.reshape(NBOND, D, D)
    k23 = k2.reshape(NBOND, D, D)
    bias1_4 = jnp.tile(conv1_bias.reshape(1, D), (1, 4))
    bias2_4 = jnp.tile(conv2_bias.reshape(1, D), (1, 4))

    y1 = _y_call(x4, k13).reshape(NBOND * NPAD, D)
    p1 = _sc_conv(y1, gidx, dsti).reshape(1, NR4, D4)
    h4, y2 = _hy_call(p1, x4, conv1_root, bias1_4, k23)
    p2 = _sc_conv(y2.reshape(NBOND * NPAD, D), gidx, dsti).reshape(1, NR4, D4)
    pooled = _final_call(p2, h4, conv2_root, bias2_4,
                         dense_W, dense_b.reshape(1, 1), ab4, segs)
    return pooled


# speedup vs baseline: 1.0905x; 1.0905x over previous
import functools

import jax
import jax.numpy as jnp
from jax import lax
from jax.experimental import pallas as pl
from jax.experimental.pallas import tpu as pltpu
from jax.experimental.pallas import tpu_sc as plsc

N = 10000
E = 160000
D = 32
NBOND = 16
NATOM = 100
G = 256

NPAD = 10240
NR4 = NPAD // 4
D4 = 4 * D

NC = 2
NS = 16
NW = NC * NS
CH = 128
K0 = 79
K1 = 0
NCHUNK = NS * (K0 + K1)
EPAD = NCHUNK * CH
TRASH = NPAD - 1
ZR = NPAD // NS

BLK4 = 512
NB4 = NR4 // BLK4

_DOT = dict(preferred_element_type=jnp.float32, precision=lax.Precision.HIGHEST)


def _bd4(w):
    z = jnp.zeros((D, D), jnp.float32)
    rows = [jnp.concatenate([w if jj == j else z for jj in range(4)], axis=1)
            for j in range(4)]
    return jnp.concatenate(rows, axis=0)


def _split_mm(v, w):
    return jnp.dot(v, _bd4(w), **_DOT)


def _prep_body(ids_ref, aemb_ref, abias_ref, bemb_ref, w1_ref, b1_ref,
               w2_ref, b2_ref, x_ref, ab_ref, k1_ref, k2_ref):
    ids = ids_ref[...]
    iota = lax.broadcasted_iota(jnp.int32, (1, NATOM), 1)
    xs, abs_ = [], []
    for j in range(4):
        oh = (ids[:, j:j + 1] == iota).astype(jnp.float32)
        xs.append(jnp.dot(oh, aemb_ref[...], **_DOT))
        abs_.append(jnp.dot(oh, abias_ref[...], **_DOT))
    x_ref[...] = jnp.concatenate(xs, axis=1)
    ab_ref[...] = jnp.concatenate(abs_, axis=1)

    @pl.when(pl.program_id(0) == 0)
    def _():
        k1_ref[...] = jnp.dot(bemb_ref[...], w1_ref[...], **_DOT) + b1_ref[...]
        k2_ref[...] = jnp.dot(bemb_ref[...], w2_ref[...], **_DOT) + b2_ref[...]


def _y_body(x_ref, k3_ref, y_ref):
    y_ref[...] = _split_mm(x_ref[...], k3_ref[0])


def _hy_body(p_ref, x_ref, root_ref, bias_ref, k3_ref, h_ref, y_ref, hs_ref):
    @pl.when(pl.program_id(0) == 0)
    def _():
        for j in range(NB4):
            sl = pl.ds(j * BLK4, BLK4)
            agg = p_ref[0, sl]
            h = jax.nn.relu(agg + _split_mm(x_ref[sl], root_ref[...])
                            + bias_ref[...])
            hs_ref[sl] = h
            h_ref[sl] = h

    y_ref[...] = _split_mm(hs_ref[...], k3_ref[0])


def _final_body(p_ref, h_ref, root_ref, bias_ref, dw_ref, db_ref, ab_ref,
                seg_ref, pool_ref):
    h = h_ref[...]
    agg = p_ref[0]
    x2 = h + jax.nn.relu(agg + _split_mm(h, root_ref[...]) + bias_ref[...])
    zc = jnp.zeros((D, 1), jnp.float32)
    dw4 = jnp.concatenate(
        [jnp.concatenate([dw_ref[...] if jj == j else zc for jj in range(4)],
                         axis=0) for j in range(4)], axis=1)
    o4 = ab_ref[...] + jnp.dot(x2, dw4, **_DOT) + db_ref[...]
    part = jnp.zeros((G, 1), jnp.float32)
    for j in range(4):
        mask = (lax.broadcasted_iota(jnp.int32, (G, BLK4), 0)
                == seg_ref[0, j]).astype(jnp.float32)
        part = part + jnp.dot(mask, o4[:, j:j + 1], **_DOT)

    @pl.when(pl.program_id(0) == 0)
    def _():
        pool_ref[...] = jnp.zeros_like(pool_ref)

    pool_ref[...] += part


NBUF = 4


def _sc_conv_body(y_hbm, gi_hbm, di_hbm, out_hbm, accum, gi_v, di_v, rows_v,
                  buf_v, semg, sems):
    c = lax.axis_index("c")
    s = lax.axis_index("s")

    @pl.when(c == 0)
    def _():
        def zrow(r, carry):
            rows_v[0, r, pl.ds(0, 16)] = jnp.zeros((16,), jnp.float32)
            rows_v[0, r, pl.ds(16, 16)] = jnp.zeros((16,), jnp.float32)
            return carry

        lax.fori_loop(0, CH, zrow, 0)
        for q in range(ZR // CH):
            pltpu.sync_copy(rows_v.at[0], accum.at[pl.ds(s * ZR + q * CH, CH)])

    plsc.subcore_barrier()

    def edge_pipeline(start, kch):
        pltpu.sync_copy(gi_hbm.at[pl.ds(start, kch)], gi_v.at[pl.ds(0, kch)])
        pltpu.sync_copy(di_hbm.at[pl.ds(start, kch)], di_v.at[pl.ds(0, kch)])

        for j in range(NBUF):
            pltpu.async_copy(y_hbm.at[gi_v.at[j]], rows_v.at[j], semg)

        def step(k, carry):
            @pl.when(jnp.logical_and(k >= 1, k - 1 + NBUF < kch))
            def _():
                pltpu.make_async_copy(rows_v.at[(k - 1) % NBUF],
                                      accum.at[di_v.at[k - 1]], sems).wait()
                pltpu.async_copy(y_hbm.at[gi_v.at[k - 1 + NBUF]],
                                 rows_v.at[(k - 1) % NBUF], semg)

            pltpu.make_async_copy(y_hbm.at[gi_v.at[k]],
                                  rows_v.at[k % NBUF], semg).wait()
            pltpu.async_copy(rows_v.at[k % NBUF], accum.at[di_v.at[k]], sems,
                             add=True)
            return carry

        lax.fori_loop(0, kch, step, 0)
        for j in range(NBUF):
            pltpu.make_async_copy(rows_v.at[j], accum.at[di_v.at[0]],
                                  sems).wait()

    @pl.when(c == 0)
    def _():
        edge_pipeline(s * K0, K0)

    plsc.subcore_barrier()

    @pl.when(c == 0)
    def _():
        pltpu.sync_copy(accum.at[pl.ds(s * ZR, ZR)], buf_v)
        pltpu.sync_copy(buf_v, out_hbm.at[0, pl.ds(s * ZR, ZR)])


_sc_conv = pl.kernel(
    _sc_conv_body,
    out_type=jax.ShapeDtypeStruct((1, NPAD, D), jnp.float32),
    mesh=plsc.VectorSubcoreMesh(core_axis_name="c", subcore_axis_name="s",
                                num_cores=NC, num_subcores=NS),
    scratch_types=[
        pltpu.VMEM_SHARED((NPAD, D), jnp.float32),
        pltpu.VMEM((K0, CH), jnp.int32),
        pltpu.VMEM((K0, CH), jnp.int32),
        pltpu.VMEM((NBUF, CH, D), jnp.float32),
        pltpu.VMEM((ZR, D), jnp.float32),
        pltpu.SemaphoreType.DMA,
        pltpu.SemaphoreType.DMA,
    ],
    compiler_params=pltpu.CompilerParams(use_tc_tiling_on_sc=False),
)


_FULL = lambda shape: pl.BlockSpec(shape, lambda *_: tuple(0 for _ in shape))


def _prep_call(ids4, atom_emb, atom_bias_emb, bond_emb, w1, b1, w2, b2):
    return pl.pallas_call(
        _prep_body,
        grid=(NB4,),
        in_specs=[
            pl.BlockSpec((BLK4, 4), lambda n: (n, 0)),
            _FULL((NATOM, D)),
            _FULL((NATOM, 1)),
            _FULL((NBOND, D)),
            _FULL((D, D * D)),
            _FULL((1, D * D)),
            _FULL((D, D * D)),
            _FULL((1, D * D)),
        ],
        out_specs=[
            pl.BlockSpec((BLK4, D4), lambda n: (n, 0)),
            pl.BlockSpec((BLK4, 4), lambda n: (n, 0)),
            _FULL((NBOND, D * D)),
            _FULL((NBOND, D * D)),
        ],
        out_shape=[
            jax.ShapeDtypeStruct((NR4, D4), jnp.float32),
            jax.ShapeDtypeStruct((NR4, 4), jnp.float32),
            jax.ShapeDtypeStruct((NBOND, D * D), jnp.float32),
            jax.ShapeDtypeStruct((NBOND, D * D), jnp.float32),
        ],
    )(ids4, atom_emb, atom_bias_emb, bond_emb, w1, b1, w2, b2)


def _y_call(x4, k3):
    return pl.pallas_call(
        _y_body,
        grid=(NBOND,),
        in_specs=[
            _FULL((NR4, D4)),
            pl.BlockSpec((1, D, D), lambda b: (b, 0, 0)),
        ],
        out_specs=pl.BlockSpec((NR4, D4), lambda b: (b, 0)),
        out_shape=jax.ShapeDtypeStruct((NBOND * NR4, D4), jnp.float32),
    )(x4, k3)


def _hy_call(p4, x4, root, bias4, k3):
    return pl.pallas_call(
        _hy_body,
        grid=(NBOND,),
        in_specs=[
            _FULL((1, NR4, D4)),
            _FULL((NR4, D4)),
            _FULL((D, D)),
            _FULL((1, D4)),
            pl.BlockSpec((1, D, D), lambda b: (b, 0, 0)),
        ],
        out_specs=[
            _FULL((NR4, D4)),
            pl.BlockSpec((NR4, D4), lambda b: (b, 0)),
        ],
        out_shape=[
            jax.ShapeDtypeStruct((NR4, D4), jnp.float32),
            jax.ShapeDtypeStruct((NBOND * NR4, D4), jnp.float32),
        ],
        scratch_shapes=[pltpu.VMEM((NR4, D4), jnp.float32)],
    )(p4, x4, root, bias4, k3)


def _final_call(p4, h4, root, bias4, dw, db, ab4, segs):
    return pl.pallas_call(
        _final_body,
        grid=(NB4,),
        in_specs=[
            pl.BlockSpec((1, BLK4, D4), lambda n: (0, n, 0)),
            pl.BlockSpec((BLK4, D4), lambda n: (n, 0)),
            _FULL((D, D)),
            _FULL((1, D4)),
            _FULL((D, 1)),
            _FULL((1, 1)),
            pl.BlockSpec((BLK4, 4), lambda n: (n, 0)),
            pl.BlockSpec((1, 4, BLK4), lambda n: (n, 0, 0)),
        ],
        out_specs=pl.BlockSpec((G, 1), lambda n: (0, 0)),
        out_shape=jax.ShapeDtypeStruct((G, 1), jnp.float32),
    )(p4, h4, root, bias4, dw, db, ab4, segs)


def kernel(x_in, edge_index, e_in, i, atom_emb, atom_bias_emb, bond_emb,
           conv1_kn_W, conv1_kn_b, conv1_root, conv1_bias,
           conv2_kn_W, conv2_kn_b, conv2_root, conv2_bias,
           dense_W, dense_b):
    src = edge_index[0].astype(jnp.int32)
    dst = edge_index[1].astype(jnp.int32)
    bond = e_in[:, 0].astype(jnp.int32)

    gidx = bond * NPAD + src
    gidx = jnp.concatenate(
        [gidx, jnp.zeros((EPAD - E,), jnp.int32)]).reshape(NCHUNK, CH)
    dsti = jnp.concatenate(
        [dst, jnp.full((EPAD - E,), TRASH, jnp.int32)]).reshape(NCHUNK, CH)

    ids4 = jnp.concatenate(
        [x_in[:, 0], jnp.zeros((NPAD - N,), x_in.dtype)]).reshape(NR4, 4)
    segs = jnp.concatenate(
        [i.astype(jnp.int32), jnp.full((NPAD - N,), G, jnp.int32)]
    ).reshape(NB4, BLK4, 4).transpose(0, 2, 1)

    x4, ab4, k1, k2 = _prep_call(ids4, atom_emb, atom_bias_emb, bond_emb,
                                 conv1_kn_W, conv1_kn_b.reshape(1, D * D),
                                 conv2_kn_W, conv2_kn_b.reshape(1, D * D))
    k13 = k1.reshape(NBOND, D, D)
    k23 = k2.reshape(NBOND, D, D)
    bias1_4 = jnp.tile(conv1_bias.reshape(1, D), (1, 4))
    bias2_4 = jnp.tile(conv2_bias.reshape(1, D), (1, 4))

    y1 = _y_call(x4, k13).reshape(NBOND * NPAD, D)
    p1 = _sc_conv(y1, gidx, dsti).reshape(1, NR4, D4)
    h4, y2 = _hy_call(p1, x4, conv1_root, bias1_4, k23)
    p2 = _sc_conv(y2.reshape(NBOND * NPAD, D), gidx, dsti).reshape(1, NR4, D4)
    pooled = _final_call(p2, h4, conv2_root, bias2_4,
                         dense_W, dense_b.reshape(1, 1), ab4, segs)
    return pooled
